# baseline (speedup 1.0000x reference)
import functools

import jax
import jax.numpy as jnp
from jax import lax
from jax.experimental import pallas as pl
from jax.experimental.pallas import tpu as pltpu
from jax.experimental.pallas import tpu_sc as plsc

_NUM_CLASSES = 100000
_BATCH = 1024
_SMOOTH = 0.1
_NEG = _SMOOTH / (_NUM_CLASSES - 1)
_POS = 1.0 - _SMOOTH

_BR = 8
_GRID = _BATCH // _BR

_NC = 2
_NS = 16
_NW = _NC * _NS
_BPW = _BATCH // _NW


def _main_body(tgt_ref, x_ref, tile_ref, tot_ref, acc_ref):
    i = pl.program_id(0)

    @pl.when(i == 0)
    def _init():
        acc_ref[0] = 0.0

    acc_ref[0] += jnp.sum(x_ref[...])

    for r in range(_BR):
        t = tgt_ref[i * _BR + r]
        start = pl.multiple_of((t // 128) * 128, 128)
        tile_ref[pl.ds(r, 1), :] = x_ref[pl.ds(r, 1), pl.ds(start, 128)]

    @pl.when(i == _GRID - 1)
    def _fini():
        tot_ref[0] = acc_ref[0]


_main = pl.pallas_call(
    _main_body,
    grid=(_GRID,),
    in_specs=[
        pl.BlockSpec(memory_space=pltpu.SMEM),
        pl.BlockSpec((_BR, _NUM_CLASSES), lambda i: (i, 0)),
    ],
    out_specs=[
        pl.BlockSpec((_BR, 128), lambda i: (i, 0)),
        pl.BlockSpec(memory_space=pltpu.SMEM),
    ],
    out_shape=[
        jax.ShapeDtypeStruct((_BATCH, 128), jnp.float32),
        jax.ShapeDtypeStruct((1,), jnp.float32),
    ],
    scratch_shapes=[pltpu.SMEM((1,), jnp.float32)],
    compiler_params=pltpu.CompilerParams(dimension_semantics=("arbitrary",)),
)


def _make_pick():
    mesh = plsc.VectorSubcoreMesh(core_axis_name="c", subcore_axis_name="s")

    @functools.partial(
        pl.kernel,
        mesh=mesh,
        out_type=jax.ShapeDtypeStruct((_BATCH,), jnp.float32),
        scratch_types=[
            pltpu.VMEM((_BPW,), jnp.int32),
            pltpu.VMEM((_BPW, 128), jnp.float32),
            pltpu.VMEM((_BPW,), jnp.float32),
        ],
    )
    def pick_kernel(y_hbm, tgt_hbm, out_hbm, col_v, buf_v, val_v):
        wid = lax.axis_index("s") * _NC + lax.axis_index("c")
        base = wid * _BPW
        pltpu.sync_copy(tgt_hbm.at[pl.ds(base, _BPW)], col_v)
        pltpu.sync_copy(y_hbm.at[pl.ds(base, _BPW), :], buf_v)
        lane_iota = lax.iota(jnp.int32, 16)
        for j in range(_BPW // 16):
            sl = pl.ds(j * 16, 16)
            cols16 = col_v[sl]
            lanes16 = cols16 % 16
            code16 = lane_iota * 128 + ((cols16 % 128) - lanes16)
            val16 = jnp.zeros((16,), jnp.float32)
            for k in range(16):
                i = j * 16 + k
                for s in range(8):
                    seg = buf_v[i, pl.ds(s * 16, 16)]
                    g = seg[lanes16]
                    val16 = jnp.where(code16 == (k * 128 + s * 16), g, val16)
            val_v[sl] = val16
        pltpu.sync_copy(val_v, out_hbm.at[pl.ds(base, _BPW)])

    return pick_kernel


_pick = _make_pick()


def _combine_body(tot_ref, p_ref, out_ref):
    g = jnp.sum(p_ref[...])
    out_ref[0] = -(_NEG * tot_ref[0] + (_POS - _NEG) * g) / _BATCH


_combine = pl.pallas_call(
    _combine_body,
    in_specs=[
        pl.BlockSpec(memory_space=pltpu.SMEM),
        pl.BlockSpec((8, 128), lambda: (0, 0)),
    ],
    out_specs=pl.BlockSpec(memory_space=pltpu.SMEM),
    out_shape=jax.ShapeDtypeStruct((1,), jnp.float32),
)


def kernel(log_softmax, target):
    tgt = target.astype(jnp.int32)
    tiles, total = _main(tgt, log_softmax)
    picked = _pick(tiles, tgt)
    out = _combine(total, picked.reshape(8, 128))
    return out[0]


# speedup vs baseline: 1.1058x; 1.1058x over previous
import functools

import jax
import jax.numpy as jnp
from jax import lax
from jax.experimental import pallas as pl
from jax.experimental.pallas import tpu as pltpu
from jax.experimental.pallas import tpu_sc as plsc

_NUM_CLASSES = 100000
_BATCH = 1024
_SMOOTH = 0.1
_NEG = _SMOOTH / (_NUM_CLASSES - 1)
_POS = 1.0 - _SMOOTH

_BR = 32
_GRID = _BATCH // _BR

_NC = 2
_NS = 16
_NW = _NC * _NS
_BPW = _BATCH // _NW


def _main_body(tgt_ref, x_ref, tile_ref, tot_ref, acc_ref):
    i = pl.program_id(0)

    @pl.when(i == 0)
    def _init():
        acc_ref[0] = 0.0

    acc_ref[0] += jnp.sum(x_ref[...])

    for r in range(_BR):
        t = tgt_ref[i * _BR + r]
        start = pl.multiple_of((t // 128) * 128, 128)
        tile_ref[pl.ds(r, 1), :] = x_ref[pl.ds(r, 1), pl.ds(start, 128)]

    @pl.when(i == _GRID - 1)
    def _fini():
        tot_ref[0] = acc_ref[0]


_main = pl.pallas_call(
    _main_body,
    grid=(_GRID,),
    in_specs=[
        pl.BlockSpec(memory_space=pltpu.SMEM),
        pl.BlockSpec((_BR, _NUM_CLASSES), lambda i: (i, 0)),
    ],
    out_specs=[
        pl.BlockSpec((_BR, 128), lambda i: (i, 0)),
        pl.BlockSpec(memory_space=pltpu.SMEM),
    ],
    out_shape=[
        jax.ShapeDtypeStruct((_BATCH, 128), jnp.float32),
        jax.ShapeDtypeStruct((1,), jnp.float32),
    ],
    scratch_shapes=[pltpu.SMEM((1,), jnp.float32)],
    compiler_params=pltpu.CompilerParams(dimension_semantics=("arbitrary",)),
)


def _make_pick():
    mesh = plsc.VectorSubcoreMesh(core_axis_name="c", subcore_axis_name="s")

    @functools.partial(
        pl.kernel,
        mesh=mesh,
        out_type=jax.ShapeDtypeStruct((_BATCH,), jnp.float32),
        scratch_types=[
            pltpu.VMEM((_BPW,), jnp.int32),
            pltpu.VMEM((_BPW, 128), jnp.float32),
            pltpu.VMEM((_BPW,), jnp.float32),
        ],
    )
    def pick_kernel(y_hbm, tgt_hbm, out_hbm, col_v, buf_v, val_v):
        wid = lax.axis_index("s") * _NC + lax.axis_index("c")
        base = wid * _BPW
        pltpu.sync_copy(tgt_hbm.at[pl.ds(base, _BPW)], col_v)
        pltpu.sync_copy(y_hbm.at[pl.ds(base, _BPW), :], buf_v)
        lane_iota = lax.iota(jnp.int32, 16)
        for j in range(_BPW // 16):
            sl = pl.ds(j * 16, 16)
            cols16 = col_v[sl]
            lanes16 = cols16 % 16
            code16 = lane_iota * 128 + ((cols16 % 128) - lanes16)
            val16 = jnp.zeros((16,), jnp.float32)
            for k in range(16):
                i = j * 16 + k
                for s in range(8):
                    seg = buf_v[i, pl.ds(s * 16, 16)]
                    g = seg[lanes16]
                    val16 = jnp.where(code16 == (k * 128 + s * 16), g, val16)
            val_v[sl] = val16
        pltpu.sync_copy(val_v, out_hbm.at[pl.ds(base, _BPW)])

    return pick_kernel


_pick = _make_pick()


def _combine_body(tot_ref, p_ref, out_ref):
    g = jnp.sum(p_ref[...])
    out_ref[0] = -(_NEG * tot_ref[0] + (_POS - _NEG) * g) / _BATCH


_combine = pl.pallas_call(
    _combine_body,
    in_specs=[
        pl.BlockSpec(memory_space=pltpu.SMEM),
        pl.BlockSpec((8, 128), lambda: (0, 0)),
    ],
    out_specs=pl.BlockSpec(memory_space=pltpu.SMEM),
    out_shape=jax.ShapeDtypeStruct((1,), jnp.float32),
)


def kernel(log_softmax, target):
    tgt = target.astype(jnp.int32)
    tiles, total = _main(tgt, log_softmax)
    picked = _pick(tiles, tgt)
    out = _combine(total, picked.reshape(8, 128))
    return out[0]


# speedup vs baseline: 1.1326x; 1.0242x over previous
import functools

import jax
import jax.numpy as jnp
from jax import lax
from jax.experimental import pallas as pl
from jax.experimental.pallas import tpu as pltpu
from jax.experimental.pallas import tpu_sc as plsc

_NUM_CLASSES = 100000
_BATCH = 1024
_SMOOTH = 0.1
_NEG = _SMOOTH / (_NUM_CLASSES - 1)
_POS = 1.0 - _SMOOTH

_BR = 16
_GRID = _BATCH // _BR

_NC = 2
_NS = 16
_NW = _NC * _NS
_BPW = _BATCH // _NW


_NBUF = 4


def _dma_in(x_hbm, bufs, sems, blk, slot):
    return pltpu.make_async_copy(
        x_hbm.at[pl.ds(blk * _BR, _BR), :], bufs.at[slot], sems.at[slot]
    )


def _main_body(tgt_ref, x_hbm, tile_ref, tot_ref, acc_ref, bufs, sems):
    i = pl.program_id(0)

    @pl.when(i == 0)
    def _init():
        acc_ref[0] = 0.0
        for k in range(_NBUF):
            _dma_in(x_hbm, bufs, sems, k, k).start()

    slot = lax.rem(i, _NBUF)
    _dma_in(x_hbm, bufs, sems, i, slot).wait()
    blk = bufs.at[slot]

    acc_ref[0] += jnp.sum(blk[...])

    for r in range(_BR):
        t = tgt_ref[i * _BR + r]
        start = pl.multiple_of((t // 128) * 128, 128)
        tile_ref[pl.ds(r, 1), :] = blk[pl.ds(r, 1), pl.ds(start, 128)]

    @pl.when(i + _NBUF < _GRID)
    def _next():
        _dma_in(x_hbm, bufs, sems, i + _NBUF, slot).start()

    @pl.when(i == _GRID - 1)
    def _fini():
        tot_ref[0] = acc_ref[0]


_main = pl.pallas_call(
    _main_body,
    grid=(_GRID,),
    in_specs=[
        pl.BlockSpec(memory_space=pltpu.SMEM),
        pl.BlockSpec(memory_space=pl.ANY),
    ],
    out_specs=[
        pl.BlockSpec((_BR, 128), lambda i: (i, 0)),
        pl.BlockSpec(memory_space=pltpu.SMEM),
    ],
    out_shape=[
        jax.ShapeDtypeStruct((_BATCH, 128), jnp.float32),
        jax.ShapeDtypeStruct((1,), jnp.float32),
    ],
    scratch_shapes=[
        pltpu.SMEM((1,), jnp.float32),
        pltpu.VMEM((_NBUF, _BR, _NUM_CLASSES), jnp.float32),
        pltpu.SemaphoreType.DMA((_NBUF,)),
    ],
    compiler_params=pltpu.CompilerParams(dimension_semantics=("arbitrary",)),
)


def _make_pick():
    mesh = plsc.VectorSubcoreMesh(core_axis_name="c", subcore_axis_name="s")

    @functools.partial(
        pl.kernel,
        mesh=mesh,
        out_type=jax.ShapeDtypeStruct((_BATCH,), jnp.float32),
        scratch_types=[
            pltpu.VMEM((_BPW,), jnp.int32),
            pltpu.VMEM((_BPW, 128), jnp.float32),
            pltpu.VMEM((_BPW,), jnp.float32),
        ],
    )
    def pick_kernel(y_hbm, tgt_hbm, out_hbm, col_v, buf_v, val_v):
        wid = lax.axis_index("s") * _NC + lax.axis_index("c")
        base = wid * _BPW
        pltpu.sync_copy(tgt_hbm.at[pl.ds(base, _BPW)], col_v)
        pltpu.sync_copy(y_hbm.at[pl.ds(base, _BPW), :], buf_v)
        lane_iota = lax.iota(jnp.int32, 16)
        for j in range(_BPW // 16):
            sl = pl.ds(j * 16, 16)
            cols16 = col_v[sl]
            lanes16 = cols16 % 16
            code16 = lane_iota * 128 + ((cols16 % 128) - lanes16)
            val16 = jnp.zeros((16,), jnp.float32)
            for k in range(16):
                i = j * 16 + k
                for s in range(8):
                    seg = buf_v[i, pl.ds(s * 16, 16)]
                    g = seg[lanes16]
                    val16 = jnp.where(code16 == (k * 128 + s * 16), g, val16)
            val_v[sl] = val16
        pltpu.sync_copy(val_v, out_hbm.at[pl.ds(base, _BPW)])

    return pick_kernel


_pick = _make_pick()


def _combine_body(tot_ref, p_ref, out_ref):
    g = jnp.sum(p_ref[...])
    out_ref[0] = -(_NEG * tot_ref[0] + (_POS - _NEG) * g) / _BATCH


_combine = pl.pallas_call(
    _combine_body,
    in_specs=[
        pl.BlockSpec(memory_space=pltpu.SMEM),
        pl.BlockSpec((8, 128), lambda: (0, 0)),
    ],
    out_specs=pl.BlockSpec(memory_space=pltpu.SMEM),
    out_shape=jax.ShapeDtypeStruct((1,), jnp.float32),
)


def kernel(log_softmax, target):
    tgt = target.astype(jnp.int32)
    tiles, total = _main(tgt, log_softmax)
    picked = _pick(tiles, tgt)
    out = _combine(total, picked.reshape(8, 128))
    return out[0]
